# Initial kernel scaffold; baseline (speedup 1.0000x reference)
import jax
import jax.numpy as jnp
from jax.experimental import pallas as pl


def kernel(x, w_padded, b_padded, other):
    raise NotImplementedError("write your pallas kernel here")


# speedup vs baseline: 1.1659x; 1.1659x over previous
import functools

import jax
import jax.numpy as jnp
from jax.experimental import pallas as pl
from jax.experimental.pallas import tpu as pltpu

IN_FEATURES = 16
OUT_FEATURES = 32
PACK = 4
K_PACKED = PACK * IN_FEATURES
N_PACKED = PACK * OUT_FEATURES
ROW_TILE = 4096


def _packed_linear_add_relu_kernel(x_ref, w_ref, b_ref, other_ref, out_ref):
    v = jnp.dot(x_ref[...], w_ref[...], preferred_element_type=jnp.float32)
    out_ref[...] = jnp.maximum(v + b_ref[...] + other_ref[...], 0.0)


@functools.partial(jax.jit, static_argnames=())
def kernel(x, w_padded, b_padded, other):
    B = x.shape[0]
    Bp = B // PACK

    w = w_padded[:, :OUT_FEATURES]
    wb = jnp.zeros((K_PACKED, N_PACKED), x.dtype)
    for j in range(PACK):
        wb = wb.at[j * IN_FEATURES:(j + 1) * IN_FEATURES,
                   j * OUT_FEATURES:(j + 1) * OUT_FEATURES].set(w)
    bb = jnp.tile(b_padded[:, :OUT_FEATURES], (1, PACK))

    xr = x.reshape(Bp, K_PACKED)
    otherr = other.reshape(Bp, N_PACKED)

    tb = min(ROW_TILE, Bp)
    grid = (pl.cdiv(Bp, tb),)

    out = pl.pallas_call(
        _packed_linear_add_relu_kernel,
        out_shape=jax.ShapeDtypeStruct((Bp, N_PACKED), x.dtype),
        grid=grid,
        in_specs=[
            pl.BlockSpec((tb, K_PACKED), lambda i: (i, 0)),
            pl.BlockSpec((K_PACKED, N_PACKED), lambda i: (0, 0)),
            pl.BlockSpec((1, N_PACKED), lambda i: (0, 0)),
            pl.BlockSpec((tb, N_PACKED), lambda i: (i, 0)),
        ],
        out_specs=pl.BlockSpec((tb, N_PACKED), lambda i: (i, 0)),
        compiler_params=pltpu.CompilerParams(
            dimension_semantics=("parallel",),
        ),
    )(xr, wb, bb, otherr)

    return out.reshape(B, OUT_FEATURES)


# speedup vs baseline: 1.3206x; 1.1327x over previous
import jax
import jax.numpy as jnp
from jax.experimental import pallas as pl
from jax.experimental.pallas import tpu as pltpu

IN_FEATURES = 16
OUT_FEATURES = 32
ROW_TILE = 4096


def _linear_add_relu_kernel(x_ref, w_ref, b_ref, other_ref, out_ref):
    w = w_ref[:, :OUT_FEATURES]
    b = b_ref[:, :OUT_FEATURES]
    v = jnp.dot(x_ref[...], w, preferred_element_type=jnp.float32)
    out_ref[...] = jnp.maximum(v + b + other_ref[...], 0.0)


@jax.jit
def kernel(x, w_padded, b_padded, other):
    B = x.shape[0]
    tb = min(ROW_TILE, B)
    grid = (pl.cdiv(B, tb),)

    return pl.pallas_call(
        _linear_add_relu_kernel,
        out_shape=jax.ShapeDtypeStruct((B, OUT_FEATURES), x.dtype),
        grid=grid,
        in_specs=[
            pl.BlockSpec((tb, IN_FEATURES), lambda i: (i, 0)),
            pl.BlockSpec((IN_FEATURES, 128), lambda i: (0, 0)),
            pl.BlockSpec((1, 128), lambda i: (0, 0)),
            pl.BlockSpec((tb, OUT_FEATURES), lambda i: (i, 0)),
        ],
        out_specs=pl.BlockSpec((tb, OUT_FEATURES), lambda i: (i, 0)),
        compiler_params=pltpu.CompilerParams(
            dimension_semantics=("parallel",),
        ),
    )(x, w_padded, b_padded, other)


# speedup vs baseline: 1.3340x; 1.0102x over previous
import jax
import jax.numpy as jnp
from jax.experimental import pallas as pl
from jax.experimental.pallas import tpu as pltpu

IN_FEATURES = 16
OUT_FEATURES = 32
ROW_TILE = 8192


def _linear_add_relu_kernel(x_ref, w_ref, b_ref, other_ref, out_ref):
    w = w_ref[:, :OUT_FEATURES]
    b = b_ref[:, :OUT_FEATURES]
    v = jnp.dot(x_ref[...], w, preferred_element_type=jnp.float32)
    out_ref[...] = jnp.maximum(v + b + other_ref[...], 0.0)


@jax.jit
def kernel(x, w_padded, b_padded, other):
    B = x.shape[0]
    tb = min(ROW_TILE, B)
    grid = (pl.cdiv(B, tb),)

    return pl.pallas_call(
        _linear_add_relu_kernel,
        out_shape=jax.ShapeDtypeStruct((B, OUT_FEATURES), x.dtype),
        grid=grid,
        in_specs=[
            pl.BlockSpec((tb, IN_FEATURES), lambda i: (i, 0)),
            pl.BlockSpec((IN_FEATURES, 128), lambda i: (0, 0)),
            pl.BlockSpec((1, 128), lambda i: (0, 0)),
            pl.BlockSpec((tb, OUT_FEATURES), lambda i: (i, 0)),
        ],
        out_specs=pl.BlockSpec((tb, OUT_FEATURES), lambda i: (i, 0)),
        compiler_params=pltpu.CompilerParams(
            dimension_semantics=("parallel",),
        ),
    )(x, w_padded, b_padded, other)
